# baseline (speedup 1.0000x reference)
import functools

import jax
import jax.numpy as jnp
from jax import lax
from jax.experimental import pallas as pl
from jax.experimental.pallas import tpu as pltpu
from jax.experimental.pallas import tpu_sc as plsc

N_NODES = 10000
IN_DIM = 128
OUT_DIM = 512
NUM_REL = 4

NUM_CORES = 2
NUM_TILES = 16
LANES = 16

CHUNK = 112
N_PAD = 10240
ROWS_PER_TILE = N_PAD // NUM_TILES
DUMMY = N_NODES
NBUF = 3
MBIT = 32768


def _sc_segment_sums(xp, packed, zrow, zcnt, ones):
    n_chunks_total = packed.shape[1]
    cpt = n_chunks_total // NUM_TILES
    assert cpt % NBUF == 0 and cpt >= 2 * NBUF

    mesh = plsc.VectorSubcoreMesh(
        core_axis_name="c", subcore_axis_name="s",
        num_cores=NUM_CORES, num_subcores=NUM_TILES)

    @functools.partial(
        pl.kernel,
        out_type=(
            jax.ShapeDtypeStruct((NUM_REL, N_PAD, IN_DIM), jnp.float32),
            jax.ShapeDtypeStruct((NUM_REL * N_PAD,), jnp.float32),
        ),
        mesh=mesh,
        scratch_types=[
            pltpu.VMEM((NBUF, 2, CHUNK), jnp.int32),
            pltpu.VMEM((NBUF, CHUNK, IN_DIM), jnp.float32),
            pltpu.VMEM((NBUF, 1, CHUNK), jnp.int32),
            pltpu.VMEM((CHUNK,), jnp.float32),
            pltpu.VMEM((ROWS_PER_TILE,), jnp.float32),
            pltpu.VMEM_SHARED((N_PAD, IN_DIM), jnp.float32),
            pltpu.VMEM_SHARED((N_PAD,), jnp.float32),
            pltpu.SemaphoreType.DMA((NBUF,)),
            pltpu.SemaphoreType.DMA((NBUF,)),
            pltpu.SemaphoreType.DMA((NBUF,)),
            pltpu.SemaphoreType.DMA((NBUF,)),
        ],
    )
    def sc_kernel(xp_h, packed_h, zrow_h, zcnt_h, ones_h, agg_h, cnt_h,
                  ebuf, rows, midx, ones_v, cbuf, agg_s, cnt_s,
                  esem, gsem, ssem, csem):
        cid = lax.axis_index("c")
        sid = lax.axis_index("s")
        row0 = sid * ROWS_PER_TILE
        chunk0 = sid * cpt
        last = cpt - 1
        pltpu.sync_copy(ones_h, ones_v)

        def fire_e(i, b):
            pltpu.async_copy(packed_h.at[rel, chunk0 + i], ebuf.at[b],
                             esem.at[b])

        def wait_e(b):
            pltpu.make_async_copy(packed_h.at[rel, chunk0], ebuf.at[b],
                                  esem.at[b]).wait()

        def fire_g(b):
            pltpu.async_copy(xp_h.at[ebuf.at[b, 0]], rows.at[b], gsem.at[b])

        def wait_g(b):
            pltpu.make_async_copy(xp_h.at[ebuf.at[b, 0]], rows.at[b],
                                  gsem.at[b]).wait()

        def fire_c(b):
            pltpu.async_copy(rows.at[b], agg_s.at[midx.at[b, 0]],
                             ssem.at[b], add=True)
            pltpu.async_copy(ones_v, cnt_s.at[midx.at[b, 0]],
                             csem.at[b], add=True)

        def wait_c(b):
            pltpu.make_async_copy(rows.at[b], agg_s.at[midx.at[b, 0]],
                                  ssem.at[b]).wait()
            pltpu.make_async_copy(ones_v, cnt_s.at[midx.at[b, 0]],
                                  csem.at[b]).wait()

        def select(b):
            for j in range(CHUNK // LANES):
                sl = pl.ds(j * LANES, LANES)
                v = ebuf[b, 1, sl]
                m = v >= MBIT
                midx[b, 0, sl] = jnp.where(m, v - MBIT, DUMMY)
                ebuf[b, 0, sl] = jnp.where(m, ebuf[b, 0, sl], 0)

        for rpass in range(NUM_REL // NUM_CORES):
            rel = cid * (NUM_REL // NUM_CORES) + rpass

            pltpu.sync_copy(zrow_h, rows.at[0])
            for k in range(ROWS_PER_TILE // CHUNK):
                pltpu.sync_copy(rows.at[0],
                                agg_s.at[pl.ds(row0 + k * CHUNK, CHUNK)])
            rem = ROWS_PER_TILE % CHUNK
            if rem:
                pltpu.sync_copy(
                    rows.at[0, pl.ds(0, rem)],
                    agg_s.at[pl.ds(row0 + ROWS_PER_TILE - rem, rem)])
            pltpu.sync_copy(zcnt_h, cbuf)
            pltpu.sync_copy(cbuf, cnt_s.at[pl.ds(row0, ROWS_PER_TILE)])
            plsc.subcore_barrier()

            for j in range(NBUF - 1):
                fire_e(j, j)
            for i in range(NBUF):
                wait_e(i)
                select(i)
                fire_g(i)
                if i >= 1:
                    wait_g(i - 1)
                    fire_c(i - 1)
                fire_e(i + NBUF - 1, (i + NBUF - 1) % NBUF)

            def body(g, carry):
                for b in range(NBUF):
                    i = NBUF * g + b
                    prev = (b - 1) % NBUF
                    wait_e(b)
                    wait_c(b)
                    select(b)
                    fire_g(b)
                    wait_g(prev)
                    fire_c(prev)
                    fire_e(jnp.minimum(i + NBUF - 1, last), prev)
                return carry

            lax.fori_loop(1, cpt // NBUF, body, 0)

            wait_g(NBUF - 1)
            fire_c(NBUF - 1)
            for b in range(NBUF):
                wait_c(b)
            for b in range(NBUF - 1):
                wait_e(b)
            plsc.subcore_barrier()

            for k in range(ROWS_PER_TILE // CHUNK):
                r0 = row0 + k * CHUNK
                pltpu.sync_copy(agg_s.at[pl.ds(r0, CHUNK)], rows.at[0])
                pltpu.sync_copy(rows.at[0], agg_h.at[rel, pl.ds(r0, CHUNK)])
            if rem:
                r0 = row0 + ROWS_PER_TILE - rem
                pltpu.sync_copy(agg_s.at[pl.ds(r0, rem)],
                                rows.at[0, pl.ds(0, rem)])
                pltpu.sync_copy(rows.at[0, pl.ds(0, rem)],
                                agg_h.at[rel, pl.ds(r0, rem)])
            pltpu.sync_copy(cnt_s.at[pl.ds(row0, ROWS_PER_TILE)], cbuf)
            pltpu.sync_copy(cbuf, cnt_h.at[pl.ds(rel * N_PAD + row0,
                                                 ROWS_PER_TILE)])
            plsc.subcore_barrier()

    return sc_kernel(xp, packed, zrow, zcnt, ones)


def _tc_combine(agg, cnt, xp, weight, root, bias2d):
    nb = 2048
    grid = N_PAD // nb

    def tc_body(agg_ref, cnt_ref, x_ref, w_ref, root_ref, bias_ref, out_ref):
        acc = jnp.dot(x_ref[...], root_ref[...],
                      preferred_element_type=jnp.float32)
        for r in range(NUM_REL):
            inv = 1.0 / jnp.clip(cnt_ref[r], 1.0)
            acc = acc + jnp.dot(agg_ref[r] * inv[:, None], w_ref[r],
                                preferred_element_type=jnp.float32)
        out_ref[...] = jnp.maximum(acc + bias_ref[...], 0.0)

    return pl.pallas_call(
        tc_body,
        grid=(grid,),
        in_specs=[
            pl.BlockSpec((NUM_REL, nb, IN_DIM), lambda i: (0, i, 0)),
            pl.BlockSpec((NUM_REL, nb), lambda i: (0, i)),
            pl.BlockSpec((nb, IN_DIM), lambda i: (i, 0)),
            pl.BlockSpec((NUM_REL, IN_DIM, OUT_DIM), lambda i: (0, 0, 0)),
            pl.BlockSpec((IN_DIM, OUT_DIM), lambda i: (0, 0)),
            pl.BlockSpec((1, OUT_DIM), lambda i: (0, 0)),
        ],
        out_specs=pl.BlockSpec((nb, OUT_DIM), lambda i: (i, 0)),
        out_shape=jax.ShapeDtypeStruct((N_PAD, OUT_DIM), jnp.float32),
    )(agg, cnt, xp, weight, root, bias2d)


def kernel(x, edge_index, edge_features, weight, root, bias):
    n, in_dim = x.shape
    e = edge_index.shape[1]
    edges_per_block = NUM_TILES * CHUNK * NBUF
    e_pad = ((e + edges_per_block - 1) // edges_per_block) * edges_per_block
    n_chunks_total = e_pad // CHUNK

    xp = jnp.pad(x, ((0, N_PAD - n), (0, 0)))
    src = jnp.pad(edge_index[0], (0, e_pad - e))
    dst = jnp.pad(edge_index[1], (0, e_pad - e))
    eft = jnp.pad(edge_features.T, ((0, 0), (0, e_pad - e)))
    packed = jnp.stack([
        jnp.broadcast_to(src, (NUM_REL, e_pad)),
        dst[None, :] + eft * MBIT,
    ], axis=1)
    packed = packed.reshape(NUM_REL, 2, n_chunks_total, CHUNK)
    packed = packed.transpose(0, 2, 1, 3)

    zrow = jnp.zeros((CHUNK, IN_DIM), jnp.float32)
    zcnt = jnp.zeros((ROWS_PER_TILE,), jnp.float32)
    ones = jnp.ones((CHUNK,), jnp.float32)

    agg, cnt = _sc_segment_sums(xp, packed, zrow, zcnt, ones)
    cnt = cnt.reshape(NUM_REL, N_PAD)
    out = _tc_combine(agg, cnt, xp, weight, root, bias.reshape(1, OUT_DIM))
    return out[:n]


# speedup vs baseline: 24.0078x; 24.0078x over previous
import functools

import jax
import jax.numpy as jnp
from jax import lax
from jax.experimental import pallas as pl
from jax.experimental.pallas import tpu as pltpu
from jax.experimental.pallas import tpu_sc as plsc

N_NODES = 10000
IN_DIM = 128
OUT_DIM = 512
NUM_REL = 4

NUM_CORES = 2
NUM_TILES = 16
LANES = 16

CHUNK = 112
N_PAD = 10240
ROWS_PER_TILE = N_PAD // NUM_TILES
DUMMY = N_NODES
NBUF = 3
MBIT = 32768


def _sc_segment_sums(xp, packed, zrow, zcnt, ones):
    n_chunks_total = packed.shape[1]
    cpt = n_chunks_total // NUM_TILES
    assert cpt % NBUF == 0 and cpt >= 2 * NBUF

    mesh = plsc.VectorSubcoreMesh(
        core_axis_name="c", subcore_axis_name="s",
        num_cores=NUM_CORES, num_subcores=NUM_TILES)

    @functools.partial(
        pl.kernel,
        out_type=(
            jax.ShapeDtypeStruct((NUM_REL, N_PAD, IN_DIM), jnp.float32),
            jax.ShapeDtypeStruct((NUM_REL * N_PAD,), jnp.float32),
        ),
        mesh=mesh,
        scratch_types=[
            pltpu.VMEM((NBUF, 2, CHUNK), jnp.int32),
            pltpu.VMEM((NBUF, CHUNK, IN_DIM), jnp.float32),
            pltpu.VMEM((NBUF, 1, CHUNK), jnp.int32),
            pltpu.VMEM((CHUNK,), jnp.float32),
            pltpu.VMEM((ROWS_PER_TILE,), jnp.float32),
            pltpu.VMEM_SHARED((N_PAD, IN_DIM), jnp.float32),
            pltpu.VMEM_SHARED((N_PAD,), jnp.float32),
            pltpu.SemaphoreType.DMA((NBUF,)),
            pltpu.SemaphoreType.DMA((NBUF,)),
            pltpu.SemaphoreType.DMA((NBUF,)),
            pltpu.SemaphoreType.DMA((NBUF,)),
        ],
    )
    def sc_kernel(xp_h, packed_h, zrow_h, zcnt_h, ones_h, agg_h, cnt_h,
                  ebuf, rows, midx, ones_v, cbuf, agg_s, cnt_s,
                  esem, gsem, ssem, csem):
        cid = lax.axis_index("c")
        sid = lax.axis_index("s")
        row0 = sid * ROWS_PER_TILE
        chunk0 = sid * cpt
        last = cpt - 1
        pltpu.sync_copy(ones_h, ones_v)

        def fire_e(i, b):
            pltpu.async_copy(packed_h.at[rel, chunk0 + i], ebuf.at[b],
                             esem.at[b])

        def wait_e(b):
            pltpu.make_async_copy(packed_h.at[rel, chunk0], ebuf.at[b],
                                  esem.at[b]).wait()

        def fire_g(b):
            pltpu.async_copy(xp_h.at[ebuf.at[b, 0]], rows.at[b], gsem.at[b])

        def wait_g(b):
            pltpu.make_async_copy(xp_h.at[ebuf.at[b, 0]], rows.at[b],
                                  gsem.at[b]).wait()

        def fire_c(b):
            pltpu.async_copy(rows.at[b], agg_s.at[midx.at[b, 0]],
                             ssem.at[b], add=True)
            pltpu.async_copy(ones_v, cnt_s.at[midx.at[b, 0]],
                             csem.at[b], add=True)

        def wait_c(b):
            pltpu.make_async_copy(rows.at[b], agg_s.at[midx.at[b, 0]],
                                  ssem.at[b]).wait()
            pltpu.make_async_copy(ones_v, cnt_s.at[midx.at[b, 0]],
                                  csem.at[b]).wait()

        def select(b):
            for j in range(CHUNK // LANES):
                sl = pl.ds(j * LANES, LANES)
                v = ebuf[b, 1, sl]
                midx[b, 0, sl] = jnp.where(v >= MBIT, v - MBIT, DUMMY)

        for rpass in range(NUM_REL // NUM_CORES):
            rel = cid * (NUM_REL // NUM_CORES) + rpass

            pltpu.sync_copy(zrow_h, rows.at[0])
            for k in range(ROWS_PER_TILE // CHUNK):
                pltpu.sync_copy(rows.at[0],
                                agg_s.at[pl.ds(row0 + k * CHUNK, CHUNK)])
            rem = ROWS_PER_TILE % CHUNK
            if rem:
                pltpu.sync_copy(
                    rows.at[0, pl.ds(0, rem)],
                    agg_s.at[pl.ds(row0 + ROWS_PER_TILE - rem, rem)])
            pltpu.sync_copy(zcnt_h, cbuf)
            pltpu.sync_copy(cbuf, cnt_s.at[pl.ds(row0, ROWS_PER_TILE)])
            plsc.subcore_barrier()

            for j in range(NBUF - 1):
                fire_e(j, j)
            for i in range(NBUF):
                wait_e(i)
                select(i)
                fire_g(i)
                if i >= 1:
                    wait_g(i - 1)
                    fire_c(i - 1)
                fire_e(i + NBUF - 1, (i + NBUF - 1) % NBUF)

            def body(g, carry):
                for b in range(NBUF):
                    i = NBUF * g + b
                    prev = (b - 1) % NBUF
                    wait_e(b)
                    wait_c(b)
                    select(b)
                    fire_g(b)
                    wait_g(prev)
                    fire_c(prev)
                    fire_e(jnp.minimum(i + NBUF - 1, last), prev)
                return carry

            lax.fori_loop(1, cpt // NBUF, body, 0)

            wait_g(NBUF - 1)
            fire_c(NBUF - 1)
            for b in range(NBUF):
                wait_c(b)
            for b in range(NBUF - 1):
                wait_e(b)
            plsc.subcore_barrier()

            for k in range(ROWS_PER_TILE // CHUNK):
                r0 = row0 + k * CHUNK
                pltpu.sync_copy(agg_s.at[pl.ds(r0, CHUNK)], rows.at[0])
                pltpu.sync_copy(rows.at[0], agg_h.at[rel, pl.ds(r0, CHUNK)])
            if rem:
                r0 = row0 + ROWS_PER_TILE - rem
                pltpu.sync_copy(agg_s.at[pl.ds(r0, rem)],
                                rows.at[0, pl.ds(0, rem)])
                pltpu.sync_copy(rows.at[0, pl.ds(0, rem)],
                                agg_h.at[rel, pl.ds(r0, rem)])
            pltpu.sync_copy(cnt_s.at[pl.ds(row0, ROWS_PER_TILE)], cbuf)
            pltpu.sync_copy(cbuf, cnt_h.at[pl.ds(rel * N_PAD + row0,
                                                 ROWS_PER_TILE)])
            plsc.subcore_barrier()

    return sc_kernel(xp, packed, zrow, zcnt, ones)


def _tc_combine(agg, cnt, xp, weight, root, bias2d):
    nb = 2048
    grid = N_PAD // nb

    def tc_body(agg_ref, cnt_ref, x_ref, w_ref, root_ref, bias_ref, out_ref):
        acc = jnp.dot(x_ref[...], root_ref[...],
                      preferred_element_type=jnp.float32)
        for r in range(NUM_REL):
            inv = 1.0 / jnp.clip(cnt_ref[r], 1.0)
            acc = acc + jnp.dot(agg_ref[r] * inv[:, None], w_ref[r],
                                preferred_element_type=jnp.float32)
        out_ref[...] = jnp.maximum(acc + bias_ref[...], 0.0)

    return pl.pallas_call(
        tc_body,
        grid=(grid,),
        in_specs=[
            pl.BlockSpec((NUM_REL, nb, IN_DIM), lambda i: (0, i, 0)),
            pl.BlockSpec((NUM_REL, nb), lambda i: (0, i)),
            pl.BlockSpec((nb, IN_DIM), lambda i: (i, 0)),
            pl.BlockSpec((NUM_REL, IN_DIM, OUT_DIM), lambda i: (0, 0, 0)),
            pl.BlockSpec((IN_DIM, OUT_DIM), lambda i: (0, 0)),
            pl.BlockSpec((1, OUT_DIM), lambda i: (0, 0)),
        ],
        out_specs=pl.BlockSpec((nb, OUT_DIM), lambda i: (i, 0)),
        out_shape=jax.ShapeDtypeStruct((N_PAD, OUT_DIM), jnp.float32),
    )(agg, cnt, xp, weight, root, bias2d)


def kernel(x, edge_index, edge_features, weight, root, bias):
    n, in_dim = x.shape
    e = edge_index.shape[1]
    edges_per_block = NUM_TILES * CHUNK * NBUF
    e_pad = ((e + edges_per_block - 1) // edges_per_block) * edges_per_block
    n_chunks_total = e_pad // CHUNK

    xp = jnp.pad(x, ((0, N_PAD - n), (0, 0)))
    src = jnp.pad(edge_index[0], (0, e_pad - e))
    dst = jnp.pad(edge_index[1], (0, e_pad - e))
    eft = jnp.pad(edge_features.T, ((0, 0), (0, e_pad - e)))
    packed = jnp.stack([
        jnp.broadcast_to(src, (NUM_REL, e_pad)),
        dst[None, :] + eft * MBIT,
    ], axis=1)
    packed = packed.reshape(NUM_REL, 2, n_chunks_total, CHUNK)
    packed = packed.transpose(0, 2, 1, 3)

    zrow = jnp.zeros((CHUNK, IN_DIM), jnp.float32)
    zcnt = jnp.zeros((ROWS_PER_TILE,), jnp.float32)
    ones = jnp.ones((CHUNK,), jnp.float32)

    agg, cnt = _sc_segment_sums(xp, packed, zrow, zcnt, ones)
    cnt = cnt.reshape(NUM_REL, N_PAD)
    out = _tc_combine(agg, cnt, xp, weight, root, bias.reshape(1, OUT_DIM))
    return out[:n]
